# baseline (speedup 1.0000x reference)
import functools
import math

import jax
import jax.numpy as jnp
from jax import lax
from jax.experimental import pallas as pl
from jax.experimental.pallas import tpu as pltpu
from jax.experimental.pallas import tpu_sc as plsc

N = 10000
DEG = 32
H = 128
NH = 4
NR = 16
DH = H // NH
E = N * DEG

S = 5
N_S = N // S
E_S = E // S

NC = 2
NS = 16
NW = NC * NS
E_PER_W = E_S // NW
CHUNK = 80
N_CHUNKS = E_PER_W // CHUNK
NBUF = 5
ROUNDS = N_CHUNKS // NBUF

BLK = 400
GRID_S = N_S // BLK


def _sc_gather(h, src_flat, s):
    mesh = plsc.VectorSubcoreMesh(core_axis_name="c", subcore_axis_name="s")

    @functools.partial(
        pl.kernel,
        mesh=mesh,
        out_type=jax.ShapeDtypeStruct((E_S, H), jnp.float32),
        scratch_types=[
            pltpu.VMEM((E_PER_W,), jnp.int32),
        ]
        + [pltpu.VMEM((CHUNK, H), jnp.float32) for _ in range(NBUF)]
        + [pltpu.SemaphoreType.DMA for _ in range(2 * NBUF)],
    )
    def gather_kernel(h_hbm, idx_hbm, out_hbm, idx_v, *bufs_sems):
        rows = bufs_sems[:NBUF]
        gsem = bufs_sems[NBUF:2 * NBUF]
        ssem = bufs_sems[2 * NBUF:]
        wid = lax.axis_index("s") * NC + lax.axis_index("c")
        wbase = wid * E_PER_W
        pltpu.sync_copy(idx_hbm.at[pl.ds(s * E_S + wbase, E_PER_W)], idx_v)

        def g_start(c, b):
            pltpu.make_async_copy(
                h_hbm.at[idx_v.at[pl.ds(c * CHUNK, CHUNK)]], rows[b], gsem[b]
            ).start()

        def g_wait(b):
            pltpu.make_async_copy(
                h_hbm.at[idx_v.at[pl.ds(0, CHUNK)]], rows[b], gsem[b]
            ).wait()

        def s_start(c, b):
            pltpu.make_async_copy(
                rows[b], out_hbm.at[pl.ds(wbase + c * CHUNK, CHUNK)], ssem[b]
            ).start()

        def s_wait(b):
            pltpu.make_async_copy(
                rows[b], out_hbm.at[pl.ds(wbase, CHUNK)], ssem[b]
            ).wait()

        for b in range(NBUF):
            g_start(b, b)

        def body(r, carry):
            for b in range(NBUF):
                c = r * NBUF + b
                g_wait(b)
                s_start(c, b)

                @pl.when(r < ROUNDS - 1)
                def _():
                    s_wait(b)
                    g_start(c + NBUF, b)
            return carry

        lax.fori_loop(0, ROUNDS, body, 0)
        for b in range(NBUF):
            s_wait(b)

    return gather_kernel(h, src_flat)


def _tc_body(h_ref, msg_ref, rel_ref, wq_ref, wk_ref, wv_ref, rv_ref, out_ref):
    eb = BLK * DEG
    hb = h_ref[...]
    msg = msg_ref[...]
    rel = rel_ref[...]

    oh = (rel == lax.broadcasted_iota(jnp.int32, (eb, NR), 1)).astype(jnp.float32)
    msg = msg + lax.dot_general(
        oh, rv_ref[...], (((1,), (0,)), ((), ())),
        preferred_element_type=jnp.float32)
    msg = jnp.where(msg >= 0, msg, 0.25 * msg)

    q = lax.dot_general(hb, wq_ref[...], (((1,), (1,)), ((), ())),
                        preferred_element_type=jnp.float32)
    kk = lax.dot_general(msg, wk_ref[...], (((1,), (1,)), ((), ())),
                         preferred_element_type=jnp.float32)
    vv = lax.dot_general(msg, wv_ref[...], (((1,), (1,)), ((), ())),
                         preferred_element_type=jnp.float32)

    Ssel = (lax.broadcasted_iota(jnp.int32, (H, NH), 0) // DH
            == lax.broadcasted_iota(jnp.int32, (H, NH), 1)).astype(jnp.float32)

    p = (kk.reshape(BLK, DEG, H) * q[:, None, :]).reshape(eb, H)
    scores = lax.dot_general(p, Ssel, (((1,), (0,)), ((), ())),
                             preferred_element_type=jnp.float32)
    s3 = scores.reshape(BLK, DEG, NH) * (1.0 / math.sqrt(DH))
    m = jnp.max(s3, axis=1, keepdims=True)
    e = jnp.exp(s3 - m)
    a = e / jnp.sum(e, axis=1, keepdims=True)

    ab = lax.dot_general(a.reshape(eb, NH), Ssel, (((1,), (1,)), ((), ())),
                         preferred_element_type=jnp.float32)
    red = jnp.sum((ab * vv).reshape(BLK, DEG, H), axis=1)

    x = jnp.where(red > 0, red, jnp.exp(red) - 1.0)
    out_ref[...] = hb + x


def _tc_attention(h, msg_s, rel_flat, Wq, Wk, Wv, relvec, s):
    blk0 = s * N_S // BLK
    return pl.pallas_call(
        _tc_body,
        grid=(GRID_S,),
        in_specs=[
            pl.BlockSpec((BLK, H), lambda i: (blk0 + i, 0)),
            pl.BlockSpec((BLK * DEG, H), lambda i: (i, 0)),
            pl.BlockSpec((BLK * DEG, 1), lambda i: (blk0 + i, 0)),
            pl.BlockSpec((H, H), lambda i: (0, 0)),
            pl.BlockSpec((H, H), lambda i: (0, 0)),
            pl.BlockSpec((H, H), lambda i: (0, 0)),
            pl.BlockSpec((NR, H), lambda i: (0, 0)),
        ],
        out_specs=pl.BlockSpec((BLK, H), lambda i: (i, 0)),
        out_shape=jax.ShapeDtypeStruct((N_S, H), jnp.float32),
    )(h, msg_s, rel_flat, Wq, Wk, Wv, relvec)


def kernel(h, src_ids, rel_ids, Wq, Wk, Wv, relvec):
    src_flat = src_ids.astype(jnp.int32).reshape(E)
    rel_flat = rel_ids.astype(jnp.int32).reshape(E, 1)
    outs = []
    for s in range(S):
        msg_s = _sc_gather(h, src_flat, s)
        outs.append(_tc_attention(h, msg_s, rel_flat, Wq, Wk, Wv, relvec, s))
    return jnp.concatenate(outs, axis=0)
